# Initial kernel scaffold; baseline (speedup 1.0000x reference)
import functools

import jax
import jax.numpy as jnp
from jax import lax
from jax.experimental import pallas as pl
from jax.experimental.pallas import tpu as pltpu
from jax.experimental.pallas import tpu_sc as plsc

_NC = 2
_NS = 16
_CHUNK = 128


def _sc_scatter_rows(h, src, dst, n_acc):
    n_rows, d = h.shape
    c_chunks = src.shape[2]
    rpt = n_acc // _NS
    mesh = plsc.VectorSubcoreMesh(core_axis_name="c", subcore_axis_name="s")

    @functools.partial(
        pl.kernel,
        mesh=mesh,
        out_type=jax.ShapeDtypeStruct((_NC, n_acc, d), jnp.float32),
        scratch_types=[
            pltpu.VMEM((c_chunks, _CHUNK), jnp.int32),
            pltpu.VMEM((c_chunks, _CHUNK), jnp.int32),
            pltpu.VMEM((_CHUNK, d), jnp.float32),
            pltpu.VMEM_SHARED((n_acc, d), jnp.float32),
            pltpu.SemaphoreType.DMA,
        ],
    )
    def k(h_hbm, src_hbm, dst_hbm, out_hbm, src_v, dst_v, rows_v, acc_sh, sem):
        cid = lax.axis_index("c")
        sid = lax.axis_index("s")

        zero16 = jnp.zeros((16,), jnp.float32)

        @pl.loop(0, _CHUNK)
        def _(r):
            @pl.loop(0, d, step=16)
            def _(j):
                rows_v[r, pl.ds(j, 16)] = zero16

        base = sid * rpt

        @pl.loop(0, rpt, step=_CHUNK)
        def _(r0):
            pltpu.sync_copy(rows_v, acc_sh.at[pl.ds(base + r0, _CHUNK)])

        pltpu.sync_copy(src_hbm.at[cid, sid], src_v)
        pltpu.sync_copy(dst_hbm.at[cid, sid], dst_v)
        plsc.subcore_barrier()

        @pl.loop(0, c_chunks)
        def _(cc):
            pltpu.async_copy(h_hbm.at[src_v.at[cc]], rows_v, sem).wait()
            pltpu.sync_copy(rows_v, acc_sh.at[dst_v.at[cc]], add=True)

        plsc.subcore_barrier()

        @pl.loop(0, rpt, step=_CHUNK)
        def _(r0):
            pltpu.sync_copy(
                acc_sh.at[pl.ds(base + r0, _CHUNK)],
                out_hbm.at[cid, pl.ds(base + r0, _CHUNK)],
            )

    return k(h, src, dst)


def _sc_degree(dst, n_acc):
    c_chunks = dst.shape[2]
    rpt = n_acc // _NS
    mesh = plsc.VectorSubcoreMesh(core_axis_name="c", subcore_axis_name="s")

    @functools.partial(
        pl.kernel,
        mesh=mesh,
        out_type=jax.ShapeDtypeStruct((_NC, n_acc, 16), jnp.float32),
        scratch_types=[
            pltpu.VMEM((c_chunks, _CHUNK), jnp.int32),
            pltpu.VMEM((_CHUNK, 16), jnp.float32),
            pltpu.VMEM((_CHUNK, 16), jnp.float32),
            pltpu.VMEM_SHARED((n_acc, 16), jnp.float32),
        ],
    )
    def k(dst_hbm, out_hbm, dst_v, ones_v, zeros_v, acc_sh):
        cid = lax.axis_index("c")
        sid = lax.axis_index("s")

        one16 = jnp.ones((16,), jnp.float32)
        zero16 = jnp.zeros((16,), jnp.float32)

        @pl.loop(0, _CHUNK)
        def _(r):
            ones_v[r, :] = one16
            zeros_v[r, :] = zero16

        base = sid * rpt

        @pl.loop(0, rpt, step=_CHUNK)
        def _(r0):
            pltpu.sync_copy(zeros_v, acc_sh.at[pl.ds(base + r0, _CHUNK)])

        pltpu.sync_copy(dst_hbm.at[cid, sid], dst_v)
        plsc.subcore_barrier()

        @pl.loop(0, c_chunks)
        def _(cc):
            pltpu.sync_copy(ones_v, acc_sh.at[dst_v.at[cc]], add=True)

        plsc.subcore_barrier()

        @pl.loop(0, rpt, step=_CHUNK)
        def _(r0):
            pltpu.sync_copy(
                acc_sh.at[pl.ds(base + r0, _CHUNK)],
                out_hbm.at[cid, pl.ds(base + r0, _CHUNK)],
            )

    return k(dst)


def _dot(a, b):
    return lax.dot_general(
        a, b, (((1,), (0,)), ((), ())),
        precision=lax.Precision.HIGHEST,
        preferred_element_type=jnp.float32,
    )


_ROWS_BLK = 1000


def _tc_matmul(x, w):
    n, kdim = x.shape
    m = w.shape[1]

    def body(x_ref, w_ref, o_ref):
        o_ref[...] = _dot(x_ref[...], w_ref[...])

    return pl.pallas_call(
        body,
        grid=(n // _ROWS_BLK,),
        in_specs=[
            pl.BlockSpec((_ROWS_BLK, kdim), lambda i: (i, 0)),
            pl.BlockSpec((kdim, m), lambda i: (0, 0)),
        ],
        out_specs=pl.BlockSpec((_ROWS_BLK, m), lambda i: (i, 0)),
        out_shape=jax.ShapeDtypeStruct((n, m), jnp.float32),
    )(x, w)


def _tc_norm_scale(p0, p1, h1):
    n, d = h1.shape

    def body(p0_ref, p1_ref, h_ref, hp_ref, dis_ref):
        deg = p0_ref[...] + p1_ref[...] + 1.0
        dis = lax.rsqrt(deg)
        dis_ref[...] = dis
        hp_ref[...] = h_ref[...] * dis

    return pl.pallas_call(
        body,
        grid=(n // _ROWS_BLK,),
        in_specs=[
            pl.BlockSpec((_ROWS_BLK, 1), lambda i: (i, 0)),
            pl.BlockSpec((_ROWS_BLK, 1), lambda i: (i, 0)),
            pl.BlockSpec((_ROWS_BLK, d), lambda i: (i, 0)),
        ],
        out_specs=[
            pl.BlockSpec((_ROWS_BLK, d), lambda i: (i, 0)),
            pl.BlockSpec((_ROWS_BLK, 1), lambda i: (i, 0)),
        ],
        out_shape=[
            jax.ShapeDtypeStruct((n, d), jnp.float32),
            jax.ShapeDtypeStruct((n, 1), jnp.float32),
        ],
    )(p0, p1, h1)


def _tc_mid(q0, q1, h1p, dis, b1, w2):
    n, d = h1p.shape
    m = w2.shape[1]

    def body(q0_ref, q1_ref, hp_ref, dis_ref, b_ref, w_ref, o_ref):
        z = (q0_ref[...] + q1_ref[...] + hp_ref[...]) * dis_ref[...] + b_ref[...]
        z = jnp.maximum(z, 0.0)
        o_ref[...] = _dot(z, w_ref[...]) * dis_ref[...]

    return pl.pallas_call(
        body,
        grid=(n // _ROWS_BLK,),
        in_specs=[
            pl.BlockSpec((_ROWS_BLK, d), lambda i: (i, 0)),
            pl.BlockSpec((_ROWS_BLK, d), lambda i: (i, 0)),
            pl.BlockSpec((_ROWS_BLK, d), lambda i: (i, 0)),
            pl.BlockSpec((_ROWS_BLK, 1), lambda i: (i, 0)),
            pl.BlockSpec((1, d), lambda i: (0, 0)),
            pl.BlockSpec((d, m), lambda i: (0, 0)),
        ],
        out_specs=pl.BlockSpec((_ROWS_BLK, m), lambda i: (i, 0)),
        out_shape=jax.ShapeDtypeStruct((n, m), jnp.float32),
    )(q0, q1, h1p, dis, b1, w2)


def _tc_final(q0, q1, h2p, dis, b2):
    n, d = h2p.shape

    def body(q0_ref, q1_ref, hp_ref, dis_ref, b_ref, o_ref):
        o_ref[...] = (
            (q0_ref[...] + q1_ref[...] + hp_ref[...]) * dis_ref[...] + b_ref[...]
        )

    return pl.pallas_call(
        body,
        grid=(n // _ROWS_BLK,),
        in_specs=[
            pl.BlockSpec((_ROWS_BLK, d), lambda i: (i, 0)),
            pl.BlockSpec((_ROWS_BLK, d), lambda i: (i, 0)),
            pl.BlockSpec((_ROWS_BLK, d), lambda i: (i, 0)),
            pl.BlockSpec((_ROWS_BLK, 1), lambda i: (i, 0)),
            pl.BlockSpec((1, d), lambda i: (0, 0)),
        ],
        out_specs=pl.BlockSpec((_ROWS_BLK, d), lambda i: (i, 0)),
        out_shape=jax.ShapeDtypeStruct((n, d), jnp.float32),
    )(q0, q1, h2p, dis, b2)


def kernel(x, edge_index, W1, b1, W2, b2):
    n, _ = x.shape
    e = edge_index.shape[1]

    src = edge_index[0].astype(jnp.int32)
    dst = edge_index[1].astype(jnp.int32)

    per_round = _NC * _NS * _CHUNK
    e_pad = -(-e // per_round) * per_round
    pad = e_pad - e
    if pad:
        src = jnp.concatenate([src, jnp.zeros((pad,), jnp.int32)])
        dst = jnp.concatenate([dst, jnp.full((pad,), n, jnp.int32)])
    c_chunks = e_pad // per_round
    src_r = src.reshape(_NC, _NS, c_chunks, _CHUNK)
    dst_r = dst.reshape(_NC, _NS, c_chunks, _CHUNK)

    slab = _NS * _CHUNK
    n_acc = -(-(n + 1) // slab) * slab

    deg_p = _sc_degree(dst_r, n_acc)
    h1 = _tc_matmul(x, W1)

    p0 = deg_p[0, :n, 0:1]
    p1 = deg_p[1, :n, 0:1]
    h1p, dis = _tc_norm_scale(p0, p1, h1)

    s1 = _sc_scatter_rows(h1p, src_r, dst_r, n_acc)
    h2p = _tc_mid(s1[0, :n], s1[1, :n], h1p, dis, b1, W2)

    s2 = _sc_scatter_rows(h2p, src_r, dst_r, n_acc)
    return _tc_final(s2[0, :n], s2[1, :n], h2p, dis, b2)


# speedup vs baseline: 12.6830x; 12.6830x over previous
import functools

import jax
import jax.numpy as jnp
from jax import lax
from jax.experimental import pallas as pl
from jax.experimental.pallas import tpu as pltpu
from jax.experimental.pallas import tpu_sc as plsc

_NC = 2
_NS = 16
_CHUNK = 128


def _sc_scatter_rows(h, src, dst, n_acc):
    n_rows, d = h.shape
    c_chunks = src.shape[2]
    rpt = n_acc // _NS
    mesh = plsc.VectorSubcoreMesh(core_axis_name="c", subcore_axis_name="s")

    @functools.partial(
        pl.kernel,
        mesh=mesh,
        out_type=jax.ShapeDtypeStruct((_NC, n_acc, d), jnp.float32),
        scratch_types=[
            pltpu.VMEM((c_chunks, _CHUNK), jnp.int32),
            pltpu.VMEM((c_chunks, _CHUNK), jnp.int32),
            pltpu.VMEM((_CHUNK, d), jnp.float32),
            pltpu.VMEM_SHARED((n_acc, d), jnp.float32),
            pltpu.SemaphoreType.DMA,
        ],
    )
    def k(h_hbm, src_hbm, dst_hbm, out_hbm, src_v, dst_v, rows_v, acc_sh, sem):
        cid = lax.axis_index("c")
        sid = lax.axis_index("s")

        zero16 = jnp.zeros((16,), jnp.float32)

        @pl.loop(0, _CHUNK)
        def _(r):
            @pl.loop(0, d, step=16)
            def _(j):
                rows_v[r, pl.ds(j, 16)] = zero16

        base = sid * rpt

        @pl.loop(0, rpt, step=_CHUNK)
        def _(r0):
            pltpu.sync_copy(rows_v, acc_sh.at[pl.ds(base + r0, _CHUNK)])

        pltpu.sync_copy(src_hbm.at[cid, sid], src_v)
        pltpu.sync_copy(dst_hbm.at[cid, sid], dst_v)
        plsc.subcore_barrier()

        @pl.loop(0, c_chunks)
        def _(cc):
            pltpu.async_copy(h_hbm.at[src_v.at[cc]], rows_v, sem).wait()
            pltpu.sync_copy(rows_v, acc_sh.at[dst_v.at[cc]], add=True)

        plsc.subcore_barrier()

        @pl.loop(0, rpt, step=_CHUNK)
        def _(r0):
            pltpu.sync_copy(
                acc_sh.at[pl.ds(base + r0, _CHUNK)],
                out_hbm.at[cid, pl.ds(base + r0, _CHUNK)],
            )

    return k(h, src, dst)


def _sc_degree(dst, n_acc):
    c_chunks = dst.shape[2]
    rpt = n_acc // _NS
    mesh = plsc.VectorSubcoreMesh(core_axis_name="c", subcore_axis_name="s")

    @functools.partial(
        pl.kernel,
        mesh=mesh,
        out_type=jax.ShapeDtypeStruct((_NC, n_acc, 16), jnp.float32),
        scratch_types=[
            pltpu.VMEM((c_chunks, _CHUNK), jnp.int32),
            pltpu.VMEM((_CHUNK, 16), jnp.float32),
            pltpu.VMEM((_CHUNK, 16), jnp.float32),
            pltpu.VMEM_SHARED((n_acc, 16), jnp.float32),
        ],
    )
    def k(dst_hbm, out_hbm, dst_v, ones_v, zeros_v, acc_sh):
        cid = lax.axis_index("c")
        sid = lax.axis_index("s")

        one16 = jnp.ones((16,), jnp.float32)
        zero16 = jnp.zeros((16,), jnp.float32)

        @pl.loop(0, _CHUNK)
        def _(r):
            ones_v[r, :] = one16
            zeros_v[r, :] = zero16

        base = sid * rpt

        @pl.loop(0, rpt, step=_CHUNK)
        def _(r0):
            pltpu.sync_copy(zeros_v, acc_sh.at[pl.ds(base + r0, _CHUNK)])

        pltpu.sync_copy(dst_hbm.at[cid, sid], dst_v)
        plsc.subcore_barrier()

        @pl.loop(0, c_chunks)
        def _(cc):
            pltpu.sync_copy(ones_v, acc_sh.at[dst_v.at[cc]], add=True)

        plsc.subcore_barrier()

        @pl.loop(0, rpt, step=_CHUNK)
        def _(r0):
            pltpu.sync_copy(
                acc_sh.at[pl.ds(base + r0, _CHUNK)],
                out_hbm.at[cid, pl.ds(base + r0, _CHUNK)],
            )

    return k(dst)


def _dot(a, b):
    return lax.dot_general(
        a, b, (((1,), (0,)), ((), ())),
        precision=lax.Precision.HIGHEST,
        preferred_element_type=jnp.float32,
    )


_ROWS_BLK = 1000


def _tc_matmul(x, w):
    n, kdim = x.shape
    m = w.shape[1]

    def body(x_ref, w_ref, o_ref):
        o_ref[...] = _dot(x_ref[...], w_ref[...])

    return pl.pallas_call(
        body,
        grid=(n // _ROWS_BLK,),
        in_specs=[
            pl.BlockSpec((_ROWS_BLK, kdim), lambda i: (i, 0)),
            pl.BlockSpec((kdim, m), lambda i: (0, 0)),
        ],
        out_specs=pl.BlockSpec((_ROWS_BLK, m), lambda i: (i, 0)),
        out_shape=jax.ShapeDtypeStruct((n, m), jnp.float32),
    )(x, w)


def _tc_norm_scale(p0, p1, h1):
    n, d = h1.shape

    def body(p0_ref, p1_ref, h_ref, hp_ref, dis_ref):
        deg = p0_ref[...] + p1_ref[...] + 1.0
        dis = lax.rsqrt(deg)
        dis_ref[...] = dis
        hp_ref[...] = h_ref[...] * dis

    return pl.pallas_call(
        body,
        grid=(n // _ROWS_BLK,),
        in_specs=[
            pl.BlockSpec((_ROWS_BLK, 1), lambda i: (i, 0)),
            pl.BlockSpec((_ROWS_BLK, 1), lambda i: (i, 0)),
            pl.BlockSpec((_ROWS_BLK, d), lambda i: (i, 0)),
        ],
        out_specs=[
            pl.BlockSpec((_ROWS_BLK, d), lambda i: (i, 0)),
            pl.BlockSpec((_ROWS_BLK, 1), lambda i: (i, 0)),
        ],
        out_shape=[
            jax.ShapeDtypeStruct((n, d), jnp.float32),
            jax.ShapeDtypeStruct((n, 1), jnp.float32),
        ],
    )(p0, p1, h1)


def _tc_mid(q0, q1, h1p, dis, b1, w2):
    n, d = h1p.shape
    m = w2.shape[1]

    def body(q0_ref, q1_ref, hp_ref, dis_ref, b_ref, w_ref, o_ref):
        z = (q0_ref[...] + q1_ref[...] + hp_ref[...]) * dis_ref[...] + b_ref[...]
        z = jnp.maximum(z, 0.0)
        o_ref[...] = _dot(z, w_ref[...]) * dis_ref[...]

    return pl.pallas_call(
        body,
        grid=(n // _ROWS_BLK,),
        in_specs=[
            pl.BlockSpec((_ROWS_BLK, d), lambda i: (i, 0)),
            pl.BlockSpec((_ROWS_BLK, d), lambda i: (i, 0)),
            pl.BlockSpec((_ROWS_BLK, d), lambda i: (i, 0)),
            pl.BlockSpec((_ROWS_BLK, 1), lambda i: (i, 0)),
            pl.BlockSpec((1, d), lambda i: (0, 0)),
            pl.BlockSpec((d, m), lambda i: (0, 0)),
        ],
        out_specs=pl.BlockSpec((_ROWS_BLK, m), lambda i: (i, 0)),
        out_shape=jax.ShapeDtypeStruct((n, m), jnp.float32),
    )(q0, q1, h1p, dis, b1, w2)


def _tc_final(q0, q1, h2p, dis, b2):
    n, d = h2p.shape

    def body(q0_ref, q1_ref, hp_ref, dis_ref, b_ref, o_ref):
        o_ref[...] = (
            (q0_ref[...] + q1_ref[...] + hp_ref[...]) * dis_ref[...] + b_ref[...]
        )

    return pl.pallas_call(
        body,
        grid=(n // _ROWS_BLK,),
        in_specs=[
            pl.BlockSpec((_ROWS_BLK, d), lambda i: (i, 0)),
            pl.BlockSpec((_ROWS_BLK, d), lambda i: (i, 0)),
            pl.BlockSpec((_ROWS_BLK, d), lambda i: (i, 0)),
            pl.BlockSpec((_ROWS_BLK, 1), lambda i: (i, 0)),
            pl.BlockSpec((1, d), lambda i: (0, 0)),
        ],
        out_specs=pl.BlockSpec((_ROWS_BLK, d), lambda i: (i, 0)),
        out_shape=jax.ShapeDtypeStruct((n, d), jnp.float32),
    )(q0, q1, h2p, dis, b2)


def kernel(x, edge_index, W1, b1, W2, b2):
    n, _ = x.shape
    e = edge_index.shape[1]

    src = edge_index[0].astype(jnp.int32)
    dst = edge_index[1].astype(jnp.int32)

    per_round = _NC * _NS * _CHUNK
    e_pad = -(-e // per_round) * per_round
    pad = e_pad - e
    if pad:
        src = jnp.concatenate([src, jnp.zeros((pad,), jnp.int32)])
        dst = jnp.concatenate([dst, jnp.full((pad,), n, jnp.int32)])
    c_chunks = e_pad // per_round
    src_r = src.reshape(_NC, _NS, c_chunks, _CHUNK)
    dst_r = dst.reshape(_NC, _NS, c_chunks, _CHUNK)

    slab = _NS * _CHUNK
    n_acc = -(-(n + 1) // slab) * slab

    deg_p = _sc_degree(dst_r, n_acc)
    h1 = _tc_matmul(x, W1)

    p0 = deg_p[0, :n, 0:1]
    p1 = deg_p[1, :n, 0:1]
    h1p, dis = _tc_norm_scale(p0, p1, h1)

    s1 = _sc_scatter_rows(h1p, src_r, dst_r, n_acc)
    h2p = _tc_mid(s1[0, :n], s1[1, :n], h1p, dis, b1.reshape(1, -1), W2)

    s2 = _sc_scatter_rows(h2p, src_r, dst_r, n_acc)
    return _tc_final(s2[0, :n], s2[1, :n], h2p, dis, b2.reshape(1, -1))
